# baseline (speedup 1.0000x reference)
import functools

import jax
import jax.numpy as jnp
from jax import lax
from jax.experimental import pallas as pl
from jax.experimental.pallas import tpu as pltpu
from jax.experimental.pallas import tpu_sc as plsc

_EPS = 1e-15
_D = 128
_N_EDGES = 320000
_TOTAL = 2 * _N_EDGES
_NC = 2
_NS = 16
_NW = _NC * _NS
_PER_W = _TOTAL // _NW
_BLK = 4000
_NBLK = _PER_W // _BLK
_SB = 80
_NSTEP = _BLK // _SB
_TSTEPS = _PER_W // _SB
_NRING = 4
_LANES = 16
_DCH = _D // _LANES

_ROWS = _TOTAL // _D
_POS_ROWS = _N_EDGES // _D


def _make_sc_logits():
    mesh = plsc.VectorSubcoreMesh(core_axis_name="c", subcore_axis_name="s")

    row_bufs = [pltpu.VMEM((2 * _SB, _D), jnp.float32)
                for _ in range(_NRING)]

    @functools.partial(
        pl.kernel,
        out_type=jax.ShapeDtypeStruct((_TOTAL,), jnp.float32),
        mesh=mesh,
        compiler_params=pltpu.CompilerParams(needs_layout_passes=False),
        scratch_types=[
            pltpu.VMEM((4 * _BLK,), jnp.int32),
            pltpu.VMEM((_BLK,), jnp.float32),
            *row_bufs,
            *([pltpu.SemaphoreType.DMA] * _NRING),
            pltpu.SemaphoreType.DMA,
        ],
    )
    def sc_logits(z_hbm, pe_hbm, ne_hbm, out_hbm,
                  idx, vals, *bufs_and_sems):
        rows = bufs_and_sems[:_NRING]
        sems = bufs_and_sems[_NRING:2 * _NRING]
        isem = bufs_and_sems[2 * _NRING]
        slots = tuple((rows[b], sems[b]) for b in range(_NRING))

        wid = lax.axis_index("s") * _NC + lax.axis_index("c")
        half = wid // _NS
        w_base = (wid % _NS) * _PER_W
        iota16 = lax.iota(jnp.int32, _LANES)
        zeros_f = jnp.zeros((_LANES,), jnp.float32)

        def _idx_off(t):
            return pl.multiple_of(
                (t // _NSTEP) % 2 * 2 * _BLK + (t % _NSTEP) * 2 * _SB,
                2 * _SB)

        def stage(bn, dst_off, sync):
            sbase = pl.multiple_of(2 * (w_base + bn * _BLK), 8)
            di = idx.at[pl.ds(dst_off, 2 * _BLK)]

            @pl.when(half == 0)
            def _():
                if sync:
                    pltpu.sync_copy(pe_hbm.at[pl.ds(sbase, 2 * _BLK)], di)
                else:
                    pltpu.async_copy(pe_hbm.at[pl.ds(sbase, 2 * _BLK)], di, isem)

            @pl.when(half == 1)
            def _():
                if sync:
                    pltpu.sync_copy(ne_hbm.at[pl.ds(sbase, 2 * _BLK)], di)
                else:
                    pltpu.async_copy(ne_hbm.at[pl.ds(sbase, 2 * _BLK)], di, isem)

        def fire(t, b):
            rb, sem = slots[b]
            off = _idx_off(t)
            pltpu.async_copy(z_hbm.at[idx.at[pl.ds(off, 2 * _SB)]], rb, sem)

        def drain(b):
            rb, sem = slots[b]
            pltpu.make_async_copy(z_hbm.at[idx.at[pl.ds(0, 2 * _SB)]], rb, sem).wait()

        def _dot_row(rb, e):
            p = [rb[2 * e, pl.ds(d * _LANES, _LANES)] *
                 rb[2 * e + 1, pl.ds(d * _LANES, _LANES)]
                 for d in range(_DCH)]
            s0 = (p[0] + p[1]) + (p[2] + p[3])
            s1 = (p[4] + p[5]) + (p[6] + p[7])
            return jnp.sum(s0 + s1)

        def compute(t, b):
            rb, _ = slots[b]
            vbase = (t % _NSTEP) * _SB

            def grp_body(g, carry):
                e0 = g * _LANES

                def quad(ii, v):
                    k0 = ii * 4
                    for k in range(4):
                        s = _dot_row(rb, e0 + k0 + k)
                        v = jnp.where(iota16 == k0 + k, s, v)
                    return v

                v = lax.fori_loop(0, _LANES // 4, quad, zeros_f)
                vals[pl.ds(vbase + e0, _LANES)] = v
                return carry

            lax.fori_loop(0, _SB // _LANES, grp_body, 0)

        def step(t, b, last_fire):
            drain(b)
            compute(t, b)
            bcur = t // _NSTEP
            tin = t % _NSTEP
            more = bcur < _NBLK - 1

            @pl.when((tin == _NSTEP // 2) & more)
            def _():
                stage(bcur + 1, pl.multiple_of((bcur + 1) % 2 * 2 * _BLK, 8),
                      sync=False)

            @pl.when((tin == _NSTEP - _NRING) & more)
            def _():
                pltpu.make_async_copy(
                    pe_hbm.at[pl.ds(0, 2 * _BLK)], idx.at[pl.ds(0, 2 * _BLK)],
                    isem).wait()

            if not last_fire:
                fire(t + _NRING, b)

            @pl.when(tin == _NSTEP - 1)
            def _():
                obase = pl.multiple_of(
                    half * _N_EDGES + w_base + bcur * _BLK, 8)
                pltpu.sync_copy(vals, out_hbm.at[pl.ds(obase, _BLK)])

        stage(0, 0, sync=True)
        for b in range(_NRING):
            fire(b, b)

        def grp(g, c):
            for b in range(_NRING):
                step(g * _NRING + b, b, last_fire=False)
            return c

        n_main = _TSTEPS // _NRING - 1
        lax.fori_loop(0, n_main, grp, 0)
        for t in range(n_main * _NRING, _TSTEPS):
            step(t, t % _NRING, last_fire=t + _NRING >= _TSTEPS)

    return sc_logits


def _loss_body(v_ref, out_ref):
    v = v_ref[...]
    p = 1.0 / (1.0 + jnp.exp(-v))
    row = lax.broadcasted_iota(jnp.int32, (_ROWS, _D), 0)
    term = jnp.where(row < _POS_ROWS,
                     -jnp.log(p + _EPS),
                     -jnp.log(1.0 - p))
    out_ref[0, 0] = jnp.sum(term) / _N_EDGES


def kernel(z, pos_edge_index, neg_edge_index):
    pe = pos_edge_index.astype(jnp.int32).T.reshape(-1)
    ne = neg_edge_index.astype(jnp.int32).T.reshape(-1)
    logits = _make_sc_logits()(z, pe, ne)
    loss = pl.pallas_call(
        _loss_body,
        out_shape=jax.ShapeDtypeStruct((1, 1), jnp.float32),
        out_specs=pl.BlockSpec(memory_space=pltpu.SMEM),
    )(logits.reshape(_ROWS, _D))
    return loss[0, 0]


# speedup vs baseline: 2.3301x; 2.3301x over previous
import functools

import jax
import jax.numpy as jnp
from jax import lax
from jax.experimental import pallas as pl
from jax.experimental.pallas import tpu as pltpu
from jax.experimental.pallas import tpu_sc as plsc

_EPS = 1e-15
_D = 128
_N_EDGES = 320000
_TOTAL = 2 * _N_EDGES
_NC = 2
_NS = 16
_NW = _NC * _NS
_PER_W = _TOTAL // _NW
_BLK = 4000
_NBLK = _PER_W // _BLK
_SB = 80
_NSTEP = _BLK // _SB
_TSTEPS = _PER_W // _SB
_NRING = 4
_LANES = 16
_DCH = _D // _LANES

_ROWS = _TOTAL // _D
_POS_ROWS = _N_EDGES // _D


def _make_sc_logits():
    mesh = plsc.VectorSubcoreMesh(core_axis_name="c", subcore_axis_name="s")

    row_bufs = [pltpu.VMEM((2 * _SB, _D), jnp.float32)
                for _ in range(_NRING)]

    @functools.partial(
        pl.kernel,
        out_type=jax.ShapeDtypeStruct((_TOTAL,), jnp.float32),
        mesh=mesh,
        compiler_params=pltpu.CompilerParams(needs_layout_passes=False),
        scratch_types=[
            pltpu.VMEM((2 * _BLK,), jnp.int32),
            pltpu.VMEM((2 * _BLK,), jnp.int32),
            pltpu.VMEM((_BLK,), jnp.float32),
            *row_bufs,
            *([pltpu.SemaphoreType.DMA] * _NRING),
            pltpu.SemaphoreType.DMA,
        ],
    )
    def sc_logits(z_hbm, pe_hbm, ne_hbm, out_hbm,
                  idx_i, idx_j, vals, *bufs_and_sems):
        rows = bufs_and_sems[:_NRING]
        sems = bufs_and_sems[_NRING:2 * _NRING]
        isem = bufs_and_sems[2 * _NRING]
        slots = tuple((rows[b], sems[b]) for b in range(_NRING))

        wid = lax.axis_index("s") * _NC + lax.axis_index("c")
        half = wid // _NS
        w_base = (wid % _NS) * _PER_W
        iota16 = lax.iota(jnp.int32, _LANES)
        zeros_f = jnp.zeros((_LANES,), jnp.float32)

        def _idx_off(t):
            return pl.multiple_of(
                (t // _NSTEP) % 2 * _BLK + (t % _NSTEP) * _SB, _SB)

        def stage(bn, dst_off, sync):
            sbase = pl.multiple_of(w_base + bn * _BLK, 8)
            di = idx_i.at[pl.ds(dst_off, _BLK)]
            dj = idx_j.at[pl.ds(dst_off, _BLK)]

            @pl.when(half == 0)
            def _():
                if sync:
                    pltpu.sync_copy(pe_hbm.at[pl.ds(sbase, _BLK)], di)
                    pltpu.sync_copy(pe_hbm.at[pl.ds(_N_EDGES + sbase, _BLK)], dj)
                else:
                    pltpu.async_copy(pe_hbm.at[pl.ds(sbase, _BLK)], di, isem)
                    pltpu.async_copy(
                        pe_hbm.at[pl.ds(_N_EDGES + sbase, _BLK)], dj, isem)

            @pl.when(half == 1)
            def _():
                if sync:
                    pltpu.sync_copy(ne_hbm.at[pl.ds(sbase, _BLK)], di)
                    pltpu.sync_copy(ne_hbm.at[pl.ds(_N_EDGES + sbase, _BLK)], dj)
                else:
                    pltpu.async_copy(ne_hbm.at[pl.ds(sbase, _BLK)], di, isem)
                    pltpu.async_copy(
                        ne_hbm.at[pl.ds(_N_EDGES + sbase, _BLK)], dj, isem)

        def fire(t, b):
            rb, sem = slots[b]
            off = _idx_off(t)
            h = _SB // 2
            pltpu.async_copy(z_hbm.at[idx_i.at[pl.ds(off, h)]],
                             rb.at[pl.ds(0, h)], sem)
            pltpu.async_copy(z_hbm.at[idx_i.at[pl.ds(off + h, h)]],
                             rb.at[pl.ds(h, h)], sem)
            pltpu.async_copy(z_hbm.at[idx_j.at[pl.ds(off, h)]],
                             rb.at[pl.ds(_SB, h)], sem)
            pltpu.async_copy(z_hbm.at[idx_j.at[pl.ds(off + h, h)]],
                             rb.at[pl.ds(_SB + h, h)], sem)

        def drain(b):
            rb, sem = slots[b]
            pltpu.make_async_copy(z_hbm.at[idx_i.at[pl.ds(0, _SB)]], rb, sem).wait()

        def _dot_row(rb, e):
            p = [rb[e, pl.ds(d * _LANES, _LANES)] *
                 rb[_SB + e, pl.ds(d * _LANES, _LANES)]
                 for d in range(_DCH)]
            s0 = (p[0] + p[1]) + (p[2] + p[3])
            s1 = (p[4] + p[5]) + (p[6] + p[7])
            return jnp.sum(s0 + s1)

        def compute(t, b):
            rb, _ = slots[b]
            vbase = (t % _NSTEP) * _SB

            def grp_body(g, carry):
                e0 = g * _LANES

                def quad(ii, v):
                    k0 = ii * 4
                    for k in range(4):
                        s = _dot_row(rb, e0 + k0 + k)
                        v = jnp.where(iota16 == k0 + k, s, v)
                    return v

                v = lax.fori_loop(0, _LANES // 4, quad, zeros_f)
                vals[pl.ds(vbase + e0, _LANES)] = v
                return carry

            lax.fori_loop(0, _SB // _LANES, grp_body, 0)

        def step(t, b, last_fire):
            drain(b)
            compute(t, b)
            bcur = t // _NSTEP
            tin = t % _NSTEP
            more = bcur < _NBLK - 1

            @pl.when((tin == _NSTEP // 2) & more)
            def _():
                stage(bcur + 1, pl.multiple_of((bcur + 1) % 2 * _BLK, 8),
                      sync=False)

            @pl.when((tin == _NSTEP - _NRING) & more)
            def _():
                pltpu.make_async_copy(
                    pe_hbm.at[pl.ds(0, _BLK)], idx_i.at[pl.ds(0, _BLK)],
                    isem).wait()
                pltpu.make_async_copy(
                    pe_hbm.at[pl.ds(0, _BLK)], idx_j.at[pl.ds(0, _BLK)],
                    isem).wait()

            if not last_fire:
                fire(t + _NRING, b)

            @pl.when(tin == _NSTEP - 1)
            def _():
                obase = pl.multiple_of(
                    half * _N_EDGES + w_base + bcur * _BLK, 8)
                pltpu.sync_copy(vals, out_hbm.at[pl.ds(obase, _BLK)])

        stage(0, 0, sync=True)
        for b in range(_NRING):
            fire(b, b)

        def grp(g, c):
            for b in range(_NRING):
                step(g * _NRING + b, b, last_fire=False)
            return c

        n_main = _TSTEPS // _NRING - 1
        lax.fori_loop(0, n_main, grp, 0)
        for t in range(n_main * _NRING, _TSTEPS):
            step(t, t % _NRING, last_fire=t + _NRING >= _TSTEPS)

    return sc_logits


def _loss_body(v_ref, out_ref):
    v = v_ref[...]
    p = 1.0 / (1.0 + jnp.exp(-v))
    row = lax.broadcasted_iota(jnp.int32, (_ROWS, _D), 0)
    term = jnp.where(row < _POS_ROWS,
                     -jnp.log(p + _EPS),
                     -jnp.log(1.0 - p))
    out_ref[0, 0] = jnp.sum(term) / _N_EDGES


def kernel(z, pos_edge_index, neg_edge_index):
    pe = pos_edge_index.astype(jnp.int32).reshape(-1)
    ne = neg_edge_index.astype(jnp.int32).reshape(-1)
    logits = _make_sc_logits()(z, pe, ne)
    loss = pl.pallas_call(
        _loss_body,
        out_shape=jax.ShapeDtypeStruct((1, 1), jnp.float32),
        out_specs=pl.BlockSpec(memory_space=pltpu.SMEM),
    )(logits.reshape(_ROWS, _D))
    return loss[0, 0]


# speedup vs baseline: 2.3360x; 1.0025x over previous
import functools

import jax
import jax.numpy as jnp
from jax import lax
from jax.experimental import pallas as pl
from jax.experimental.pallas import tpu as pltpu
from jax.experimental.pallas import tpu_sc as plsc

_EPS = 1e-15
_D = 128
_N_EDGES = 320000
_TOTAL = 2 * _N_EDGES
_NC = 2
_NS = 16
_NW = _NC * _NS
_PER_W = _TOTAL // _NW
_BLK = 4000
_NBLK = _PER_W // _BLK
_SB = 80
_NSTEP = _BLK // _SB
_TSTEPS = _PER_W // _SB
_NRING = 4
_LANES = 16
_DCH = _D // _LANES

_ROWS = _TOTAL // _D
_POS_ROWS = _N_EDGES // _D


def _make_sc_logits():
    mesh = plsc.VectorSubcoreMesh(core_axis_name="c", subcore_axis_name="s")

    row_bufs = [pltpu.VMEM((2 * _SB, _D), jnp.float32)
                for _ in range(_NRING)]

    @functools.partial(
        pl.kernel,
        out_type=jax.ShapeDtypeStruct((_TOTAL,), jnp.float32),
        mesh=mesh,
        compiler_params=pltpu.CompilerParams(needs_layout_passes=False),
        scratch_types=[
            pltpu.VMEM((2 * _BLK,), jnp.int32),
            pltpu.VMEM((2 * _BLK,), jnp.int32),
            pltpu.VMEM((_BLK,), jnp.float32),
            *row_bufs,
            *([pltpu.SemaphoreType.DMA] * _NRING),
            pltpu.SemaphoreType.DMA,
        ],
    )
    def sc_logits(z_hbm, pe_hbm, ne_hbm, out_hbm,
                  idx_i, idx_j, vals, *bufs_and_sems):
        rows = bufs_and_sems[:_NRING]
        sems = bufs_and_sems[_NRING:2 * _NRING]
        isem = bufs_and_sems[2 * _NRING]
        slots = tuple((rows[b], sems[b]) for b in range(_NRING))

        wid = lax.axis_index("s") * _NC + lax.axis_index("c")
        half = wid // _NS
        w_base = (wid % _NS) * _PER_W
        iota16 = lax.iota(jnp.int32, _LANES)
        zeros_f = jnp.zeros((_LANES,), jnp.float32)

        def _idx_off(t):
            return pl.multiple_of(
                (t // _NSTEP) % 2 * _BLK + (t % _NSTEP) * _SB, _SB)

        def stage(bn, dst_off, sync):
            sbase = pl.multiple_of(w_base + bn * _BLK, 8)
            di = idx_i.at[pl.ds(dst_off, _BLK)]
            dj = idx_j.at[pl.ds(dst_off, _BLK)]

            @pl.when(half == 0)
            def _():
                if sync:
                    pltpu.sync_copy(pe_hbm.at[pl.ds(sbase, _BLK)], di)
                    pltpu.sync_copy(pe_hbm.at[pl.ds(_N_EDGES + sbase, _BLK)], dj)
                else:
                    pltpu.async_copy(pe_hbm.at[pl.ds(sbase, _BLK)], di, isem)
                    pltpu.async_copy(
                        pe_hbm.at[pl.ds(_N_EDGES + sbase, _BLK)], dj, isem)

            @pl.when(half == 1)
            def _():
                if sync:
                    pltpu.sync_copy(ne_hbm.at[pl.ds(sbase, _BLK)], di)
                    pltpu.sync_copy(ne_hbm.at[pl.ds(_N_EDGES + sbase, _BLK)], dj)
                else:
                    pltpu.async_copy(ne_hbm.at[pl.ds(sbase, _BLK)], di, isem)
                    pltpu.async_copy(
                        ne_hbm.at[pl.ds(_N_EDGES + sbase, _BLK)], dj, isem)

        def fire(t, b):
            rb, sem = slots[b]
            off = _idx_off(t)
            pltpu.async_copy(z_hbm.at[idx_i.at[pl.ds(off, _SB)]],
                             rb.at[pl.ds(0, _SB)], sem)
            pltpu.async_copy(z_hbm.at[idx_j.at[pl.ds(off, _SB)]],
                             rb.at[pl.ds(_SB, _SB)], sem)

        def drain(b):
            rb, sem = slots[b]
            pltpu.make_async_copy(z_hbm.at[idx_i.at[pl.ds(0, _SB)]], rb, sem).wait()

        def _dot_row(rb, e):
            p = [rb[e, pl.ds(d * _LANES, _LANES)] *
                 rb[_SB + e, pl.ds(d * _LANES, _LANES)]
                 for d in range(_DCH)]
            s0 = (p[0] + p[1]) + (p[2] + p[3])
            s1 = (p[4] + p[5]) + (p[6] + p[7])
            return jnp.sum(s0 + s1)

        def compute(t, b):
            rb, _ = slots[b]
            vbase = (t % _NSTEP) * _SB

            def grp_body(g, carry):
                e0 = g * _LANES

                def quad(ii, v):
                    k0 = ii * 4
                    for k in range(4):
                        s = _dot_row(rb, e0 + k0 + k)
                        v = jnp.where(iota16 == k0 + k, s, v)
                    return v

                v = lax.fori_loop(0, _LANES // 4, quad, zeros_f)
                vals[pl.ds(vbase + e0, _LANES)] = v
                return carry

            lax.fori_loop(0, _SB // _LANES, grp_body, 0)

        def step(t, b, last_fire):
            drain(b)
            compute(t, b)
            bcur = t // _NSTEP
            tin = t % _NSTEP
            more = bcur < _NBLK - 1

            @pl.when((tin == _NSTEP // 2) & more)
            def _():
                stage(bcur + 1, pl.multiple_of((bcur + 1) % 2 * _BLK, 8),
                      sync=False)

            @pl.when((tin == _NSTEP - _NRING) & more)
            def _():
                pltpu.make_async_copy(
                    pe_hbm.at[pl.ds(0, _BLK)], idx_i.at[pl.ds(0, _BLK)],
                    isem).wait()
                pltpu.make_async_copy(
                    pe_hbm.at[pl.ds(0, _BLK)], idx_j.at[pl.ds(0, _BLK)],
                    isem).wait()

            if not last_fire:
                fire(t + _NRING, b)

            @pl.when(tin == _NSTEP - 1)
            def _():
                obase = pl.multiple_of(
                    half * _N_EDGES + w_base + bcur * _BLK, 8)
                pltpu.sync_copy(vals, out_hbm.at[pl.ds(obase, _BLK)])

        stage(0, 0, sync=True)
        for b in range(_NRING):
            fire(b, b)

        def grp(g, c):
            for b in range(_NRING):
                step(g * _NRING + b, b, last_fire=False)
            return c

        n_main = _TSTEPS // _NRING - 1
        lax.fori_loop(0, n_main, grp, 0)
        for t in range(n_main * _NRING, _TSTEPS):
            step(t, t % _NRING, last_fire=t + _NRING >= _TSTEPS)

    return sc_logits


def _loss_body(v_ref, out_ref):
    v = v_ref[...]
    p = 1.0 / (1.0 + jnp.exp(-v))
    row = lax.broadcasted_iota(jnp.int32, (_ROWS, _D), 0)
    term = jnp.where(row < _POS_ROWS,
                     -jnp.log(p + _EPS),
                     -jnp.log(1.0 - p))
    out_ref[0, 0] = jnp.sum(term) / _N_EDGES


def kernel(z, pos_edge_index, neg_edge_index):
    pe = pos_edge_index.astype(jnp.int32).reshape(-1)
    ne = neg_edge_index.astype(jnp.int32).reshape(-1)
    logits = _make_sc_logits()(z, pe, ne)
    loss = pl.pallas_call(
        _loss_body,
        out_shape=jax.ShapeDtypeStruct((1, 1), jnp.float32),
        out_specs=pl.BlockSpec(memory_space=pltpu.SMEM),
    )(logits.reshape(_ROWS, _D))
    return loss[0, 0]
